# Initial kernel scaffold; baseline (speedup 1.0000x reference)
import jax
import jax.numpy as jnp
from jax.experimental import pallas as pl


def kernel(x, edge_index, Wl1, bl1, Wr1, Wl2, bl2, Wr2):
    raise NotImplementedError("write your pallas kernel here")


# speedup vs baseline: 3.3972x; 3.3972x over previous
import jax
import jax.numpy as jnp
from jax import lax
from jax.experimental import pallas as pl
from jax.experimental.pallas import tpu as pltpu
from jax.experimental.pallas import tpu_sc as plsc

N = 10000
D = 128
E = 320000

NC = 2
NS = 16
NW = NC * NS
C = 128
EPW = 10240
CH = EPW // C
EP = NW * EPW
NP = N + 8
IB = 8
RB = 624

_mesh = plsc.VectorSubcoreMesh(core_axis_name="c", subcore_axis_name="s")


def _zero_acc(zsrc_hbm, acc, s):
    pltpu.sync_copy(zsrc_hbm.at[pl.ds(s * RB, RB)], acc.at[pl.ds(s * RB, RB)])

    @pl.when(s == NS - 1)
    def _():
        t0 = NS * RB
        pltpu.sync_copy(zsrc_hbm.at[pl.ds(t0, NP - t0)], acc.at[pl.ds(t0, NP - t0)])


def _copy_out(acc, out_hbm, c, s):
    pltpu.sync_copy(acc.at[pl.ds(s * RB, RB)], out_hbm.at[c, pl.ds(s * RB, RB)])

    @pl.when(s == NS - 1)
    def _():
        t0 = NS * RB
        pltpu.sync_copy(acc.at[pl.ds(t0, N - t0)], out_hbm.at[c, pl.ds(t0, N - t0)])


def _sc_agg_body(x_hbm, src_hbm, dst_hbm, zacc_hbm, out_hbm,
                 srcv, dstv, rows, acc, sem):
    c = lax.axis_index("c")
    s = lax.axis_index("s")
    w = s * NC + c
    _zero_acc(zacc_hbm, acc, s)
    plsc.subcore_barrier()

    def outer(g, carry):
        pltpu.sync_copy(src_hbm.at[w, pl.ds(g * IB, IB)], srcv)
        pltpu.sync_copy(dst_hbm.at[w, pl.ds(g * IB, IB)], dstv)

        def step(j, inner):
            pltpu.async_copy(x_hbm.at[srcv.at[j]], rows, sem).wait()
            pltpu.sync_copy(rows, acc.at[dstv.at[j]], add=True)
            return inner

        return lax.fori_loop(0, IB, step, carry)

    lax.fori_loop(0, CH // IB, outer, 0)
    plsc.subcore_barrier()
    _copy_out(acc, out_hbm, c, s)


_sc_agg = pl.kernel(
    _sc_agg_body,
    out_type=jax.ShapeDtypeStruct((NC, N, D), jnp.float32),
    mesh=_mesh,
    scratch_types=[
        pltpu.VMEM((IB, C), jnp.int32),
        pltpu.VMEM((IB, C), jnp.int32),
        pltpu.VMEM((C, D), jnp.float32),
        pltpu.VMEM_SHARED((NP, D), jnp.float32),
        pltpu.SemaphoreType.DMA,
    ],
)


def _sc_count_body(dst_hbm, zacc_hbm, ones_hbm, out_hbm, dstv, onesv, acc):
    c = lax.axis_index("c")
    s = lax.axis_index("s")
    w = s * NC + c
    _zero_acc(zacc_hbm, acc, s)
    pltpu.sync_copy(ones_hbm, onesv)
    plsc.subcore_barrier()

    def outer(g, carry):
        pltpu.sync_copy(dst_hbm.at[w, pl.ds(g * IB, IB)], dstv)

        def step(j, inner):
            pltpu.sync_copy(onesv, acc.at[dstv.at[j]], add=True)
            return inner

        return lax.fori_loop(0, IB, step, carry)

    lax.fori_loop(0, CH // IB, outer, 0)
    plsc.subcore_barrier()
    _copy_out(acc, out_hbm, c, s)


_sc_count = pl.kernel(
    _sc_count_body,
    out_type=jax.ShapeDtypeStruct((NC, N, D), jnp.float32),
    mesh=_mesh,
    scratch_types=[
        pltpu.VMEM((IB, C), jnp.int32),
        pltpu.VMEM((C, D), jnp.float32),
        pltpu.VMEM_SHARED((NP, D), jnp.float32),
    ],
)


def _tc_layer_body(part_ref, cntp_ref, x_ref, wl_ref, bl_ref, wr_ref, relu_ref, o_ref):
    agg = part_ref[0] + part_ref[1]
    cnt = cntp_ref[0, :, :1] + cntp_ref[1, :, :1]
    mean = agg / jnp.maximum(cnt, 1.0)
    y = (lax.dot_general(mean, wl_ref[...], (((1,), (1,)), ((), ())),
                         preferred_element_type=jnp.float32)
         + bl_ref[...]
         + lax.dot_general(x_ref[...], wr_ref[...], (((1,), (1,)), ((), ())),
                           preferred_element_type=jnp.float32))
    o_ref[...] = jnp.where(relu_ref[0, 0] > 0, jnp.maximum(y, 0.0), y)


def _tc_layer(relu, part, cntp, x, wl, bl, wr):
    flag = jnp.full((1, 1), 1.0 if relu else 0.0, jnp.float32)
    return pl.pallas_call(
        _tc_layer_body,
        out_shape=jax.ShapeDtypeStruct((N, D), jnp.float32),
    )(part, cntp, x, wl, bl, wr, flag)


def kernel(x, edge_index, Wl1, bl1, Wr1, Wl2, bl2, Wr2):
    src = edge_index[0].astype(jnp.int32)
    dst = edge_index[1].astype(jnp.int32)
    pad = EP - E
    srcp = jnp.concatenate([src, jnp.zeros((pad,), jnp.int32)]).reshape(NW, CH, C)
    dstp = jnp.concatenate([dst, jnp.full((pad,), N, jnp.int32)]).reshape(NW, CH, C)
    zacc = jnp.zeros((NP, D), jnp.float32)
    ones = jnp.ones((C, D), jnp.float32)

    cntp = _sc_count(dstp, zacc, ones)
    part1 = _sc_agg(x, srcp, dstp, zacc)
    h = _tc_layer(True, part1, cntp, x, Wl1, bl1.reshape(1, D), Wr1)
    part2 = _sc_agg(h, srcp, dstp, zacc)
    out = _tc_layer(False, part2, cntp, h, Wl2, bl2.reshape(1, D), Wr2)
    return out
